# baseline (speedup 1.0000x reference)
import functools

import jax
import jax.numpy as jnp
from jax import lax
from jax.experimental import pallas as pl
from jax.experimental.pallas import tpu as pltpu
from jax.experimental.pallas import tpu_sc as plsc

N_NODES = 10000
D = 128

NC = 2
NS = 16

CHUNK = 128
CH = 80
E_PAD = NC * NS * CH * CHUNK
IBLK = 16
NIB_C = (10, 0)
CH_C = (NIB_C[0] * IBLK, max(NIB_C[1], 1) * IBLK)
NIB_DEG = 5

ACC_ROWS = 10240
DUMP_ROW = N_NODES
ACC_PER_SUB = ACC_ROWS // NS

_mesh = lambda: plsc.VectorSubcoreMesh(core_axis_name="c", subcore_axis_name="s")


def _fill_vmem(buf, n_rows, width, value):
    def row(i, carry):
        for k in range(width // 16):
            buf[i, pl.ds(k * 16, 16)] = jnp.full((16,), value, jnp.float32)
        return carry
    lax.fori_loop(0, n_rows, row, 0)


def _zero_vmem(buf, n_rows, width):
    _fill_vmem(buf, n_rows, width, 0.0)


NBUF = 2


def _agg_body(p_hbm, src_hbm, dst_hbm, out_hbm,
              src_v, dst_v, rows0, rows1, agg_sh,
              gs0, gs1, ss0, ss1):
    c = lax.axis_index("c")
    s = lax.axis_index("s")
    bufs = (rows0, rows1)
    gsem = (gs0, gs1)
    ssem = (ss0, ss1)

    _zero_vmem(rows0, CHUNK, D)
    zbase = s * ACC_PER_SUB
    def zcp(j, carry):
        pltpu.sync_copy(rows0, agg_sh.at[pl.ds(zbase + j * CHUNK, CHUNK)])
        return carry
    lax.fori_loop(0, ACC_PER_SUB // CHUNK, zcp, 0)
    plsc.subcore_barrier()

    nib = jnp.where(c == 0, NIB_C[0], NIB_C[1])
    def blk(b, carry):
        pltpu.sync_copy(src_hbm.at[c, s, pl.ds(b * IBLK, IBLK)], src_v)
        pltpu.sync_copy(dst_hbm.at[c, s, pl.ds(b * IBLK, IBLK)], dst_v)

        for k in range(NBUF):
            pltpu.async_copy(p_hbm.at[src_v.at[k]], bufs[k], gsem[k])

        def step(jj, carry2):
            base = jj * NBUF
            for k in range(NBUF):
                pltpu.make_async_copy(p_hbm.at[src_v.at[base + k]],
                                      bufs[k], gsem[k]).wait()
                pltpu.async_copy(bufs[k], agg_sh.at[dst_v.at[base + k]],
                                 ssem[k], add=True)
            for k in range(NBUF):
                nxt = base + NBUF + k
                pltpu.make_async_copy(bufs[k], agg_sh.at[dst_v.at[base + k]],
                                      ssem[k]).wait()
                pltpu.async_copy(p_hbm.at[src_v.at[nxt]], bufs[k], gsem[k])
            return carry2
        lax.fori_loop(0, IBLK // NBUF - 1, step, 0)

        last = IBLK - NBUF
        for k in range(NBUF):
            pltpu.make_async_copy(p_hbm.at[src_v.at[last + k]],
                                  bufs[k], gsem[k]).wait()
            pltpu.async_copy(bufs[k], agg_sh.at[dst_v.at[last + k]],
                             ssem[k], add=True)
        for k in range(NBUF):
            pltpu.make_async_copy(bufs[k], agg_sh.at[dst_v.at[last + k]],
                                  ssem[k]).wait()
        return carry
    lax.fori_loop(0, nib, blk, 0)

    plsc.subcore_barrier()

    def ocp(j, carry):
        r0 = zbase + j * CHUNK
        pltpu.sync_copy(agg_sh.at[pl.ds(r0, CHUNK)], rows0)
        pltpu.sync_copy(rows0, out_hbm.at[c, pl.ds(r0, CHUNK)])
        return carry
    lax.fori_loop(0, ACC_PER_SUB // CHUNK, ocp, 0)


def _sc_aggregate(p, src_r, dst_r):
    return pl.kernel(
        _agg_body,
        out_type=jax.ShapeDtypeStruct((NC, ACC_ROWS, D), jnp.float32),
        mesh=_mesh(),
        scratch_types=[
            pltpu.VMEM((IBLK, CHUNK), jnp.int32),
            pltpu.VMEM((IBLK, CHUNK), jnp.int32),
            pltpu.VMEM((CHUNK, D), jnp.float32),
            pltpu.VMEM((CHUNK, D), jnp.float32),
            pltpu.VMEM_SHARED((ACC_ROWS, D), jnp.float32),
            pltpu.SemaphoreType.DMA,
            pltpu.SemaphoreType.DMA,
            pltpu.SemaphoreType.DMA,
            pltpu.SemaphoreType.DMA,
        ],
    )(p, src_r, dst_r)


def _deg_body(dst_hbm, out_hbm, dst_v, ones_v, agg_sh):
    c = lax.axis_index("c")
    s = lax.axis_index("s")

    _zero_vmem(ones_v, CHUNK, D)
    zbase = s * ACC_PER_SUB
    def zcp(j, carry):
        pltpu.sync_copy(ones_v, agg_sh.at[pl.ds(zbase + j * CHUNK, CHUNK)])
        return carry
    lax.fori_loop(0, ACC_PER_SUB // CHUNK, zcp, 0)

    _fill_vmem(ones_v, CHUNK, D, 1.0)
    plsc.subcore_barrier()

    def blk(b, carry):
        pltpu.sync_copy(dst_hbm.at[c, s, pl.ds(b * IBLK, IBLK)], dst_v)
        def step(j, carry2):
            pltpu.sync_copy(ones_v, agg_sh.at[dst_v.at[j]], add=True)
            return carry2
        lax.fori_loop(0, IBLK, step, 0)
        return carry
    lax.fori_loop(0, NIB_DEG, blk, 0)

    plsc.subcore_barrier()

    def ocp(j, carry):
        r0 = zbase + j * CHUNK
        pltpu.sync_copy(agg_sh.at[pl.ds(r0, CHUNK)], ones_v)
        pltpu.sync_copy(ones_v, out_hbm.at[c, pl.ds(r0, CHUNK)])
        return carry
    lax.fori_loop(0, ACC_PER_SUB // CHUNK, ocp, 0)


def _sc_degrees(dst_r):
    return pl.kernel(
        _deg_body,
        out_type=jax.ShapeDtypeStruct((NC, ACC_ROWS, D), jnp.float32),
        mesh=_mesh(),
        scratch_types=[
            pltpu.VMEM((IBLK, CHUNK), jnp.int32),
            pltpu.VMEM((CHUNK, D), jnp.float32),
            pltpu.VMEM_SHARED((ACC_ROWS, D), jnp.float32),
        ],
    )(dst_r)


RB = 2000
GRID = N_NODES // RB


def _matmul_body(h_ref, w_ref, o_ref):
    o_ref[...] = jnp.dot(h_ref[...], w_ref[...],
                         preferred_element_type=jnp.float32)


def _tc_matmul(h, w):
    return pl.pallas_call(
        _matmul_body,
        grid=(GRID,),
        in_specs=[
            pl.BlockSpec((RB, D), lambda i: (i, 0)),
            pl.BlockSpec((D, D), lambda i: (0, 0)),
        ],
        out_specs=pl.BlockSpec((RB, D), lambda i: (i, 0)),
        out_shape=jax.ShapeDtypeStruct((N_NODES, D), jnp.float32),
    )(h, w)


def _combine_body(apply_relu, h_ref, w_ref, b_ref, parts_ref, deg_ref, o_ref):
    agg = parts_ref[0] + parts_ref[1]
    deg = deg_ref[0, :, 0:1] + deg_ref[1, :, 0:1]
    inv = 1.0 / jnp.maximum(deg, 1.0)
    out = jnp.dot(h_ref[...], w_ref[...],
                  preferred_element_type=jnp.float32)
    out = out + b_ref[...] + inv * agg
    if apply_relu:
        out = jnp.maximum(out, 0.0)
    o_ref[...] = out


def _tc_combine(h, w_self, b, parts, deg_parts, apply_relu):
    return pl.pallas_call(
        functools.partial(_combine_body, apply_relu),
        grid=(GRID,),
        in_specs=[
            pl.BlockSpec((RB, D), lambda i: (i, 0)),
            pl.BlockSpec((D, D), lambda i: (0, 0)),
            pl.BlockSpec((1, D), lambda i: (0, 0)),
            pl.BlockSpec((NC, RB, D), lambda i: (0, i, 0)),
            pl.BlockSpec((NC, RB, D), lambda i: (0, i, 0)),
        ],
        out_specs=pl.BlockSpec((RB, D), lambda i: (i, 0)),
        out_shape=jax.ShapeDtypeStruct((N_NODES, D), jnp.float32),
    )(h, w_self, b.reshape(1, D), parts, deg_parts)


def kernel(x, edge_index, W_self0, W_neigh0, b0, W_self1, W_neigh1, b1,
           W_self2, W_neigh2, b2):
    src = edge_index[0].astype(jnp.int32)
    dst = edge_index[1].astype(jnp.int32)
    pad = E_PAD - src.shape[0]
    arange_pad = jnp.arange(pad, dtype=jnp.int32) % CHUNK
    src_p = jnp.concatenate([src, jnp.zeros((pad,), jnp.int32)])
    dst_p = jnp.concatenate([dst, DUMP_ROW + arange_pad])

    n0 = NS * CH_C[0] * CHUNK
    ch1 = NIB_C[1] * IBLK
    filler_i = jnp.zeros((NS, CH_C[0] - ch1, CHUNK), jnp.int32)
    filler_d = jnp.full((NS, CH_C[0] - ch1, CHUNK), DUMP_ROW, jnp.int32)
    src_r = jnp.stack([
        src_p[:n0].reshape(NS, CH_C[0], CHUNK),
        jnp.concatenate(
            [src_p[n0:].reshape(NS, ch1, CHUNK), filler_i], axis=1)])
    dst_r = jnp.stack([
        dst_p[:n0].reshape(NS, CH_C[0], CHUNK),
        jnp.concatenate(
            [dst_p[n0:].reshape(NS, ch1, CHUNK), filler_d], axis=1)])

    dst_deg = dst_p.reshape(NC, NS, CH, CHUNK)
    deg_parts = _sc_degrees(dst_deg)

    h = x
    for (w_s, w_n, b, relu) in (
            (W_self0, W_neigh0, b0, True),
            (W_self1, W_neigh1, b1, True),
            (W_self2, W_neigh2, b2, False)):
        p = _tc_matmul(h, w_n)
        parts = _sc_aggregate(p, src_r, dst_r)
        h = _tc_combine(h, w_s, b, parts, deg_parts, relu)
    return h


# speedup vs baseline: 1.3892x; 1.3892x over previous
import functools

import jax
import jax.numpy as jnp
from jax import lax
from jax.experimental import pallas as pl
from jax.experimental.pallas import tpu as pltpu
from jax.experimental.pallas import tpu_sc as plsc

N_NODES = 10000
D = 128

NC = 2
NS = 16

CHUNK = 128
CH = 80
E_PAD = NC * NS * CH * CHUNK
IBLK = 16
NIB_C = (8, 2)
CH_C = (NIB_C[0] * IBLK, max(NIB_C[1], 1) * IBLK)
NIB_DEG = 5

ACC_ROWS = 10240
DUMP_ROW = N_NODES
ACC_PER_SUB = ACC_ROWS // NS

_mesh = lambda: plsc.VectorSubcoreMesh(core_axis_name="c", subcore_axis_name="s")


def _fill_vmem(buf, n_rows, width, value):
    def row(i, carry):
        for k in range(width // 16):
            buf[i, pl.ds(k * 16, 16)] = jnp.full((16,), value, jnp.float32)
        return carry
    lax.fori_loop(0, n_rows, row, 0)


def _zero_vmem(buf, n_rows, width):
    _fill_vmem(buf, n_rows, width, 0.0)


NBUF = 2


def _agg_body(p_hbm, src_hbm, dst_hbm, out_hbm,
              src_v, dst_v, rows0, rows1, agg_sh,
              gs0, gs1, ss0, ss1):
    c = lax.axis_index("c")
    s = lax.axis_index("s")
    bufs = (rows0, rows1)
    gsem = (gs0, gs1)
    ssem = (ss0, ss1)

    _zero_vmem(rows0, CHUNK, D)
    zbase = s * ACC_PER_SUB
    def zcp(j, carry):
        pltpu.sync_copy(rows0, agg_sh.at[pl.ds(zbase + j * CHUNK, CHUNK)])
        return carry
    lax.fori_loop(0, ACC_PER_SUB // CHUNK, zcp, 0)
    plsc.subcore_barrier()

    nib = jnp.where(c == 0, NIB_C[0], NIB_C[1])
    def blk(b, carry):
        pltpu.sync_copy(src_hbm.at[c, s, pl.ds(b * IBLK, IBLK)], src_v)
        pltpu.sync_copy(dst_hbm.at[c, s, pl.ds(b * IBLK, IBLK)], dst_v)

        for k in range(NBUF):
            pltpu.async_copy(p_hbm.at[src_v.at[k]], bufs[k], gsem[k])

        def step(jj, carry2):
            base = jj * NBUF
            for k in range(NBUF):
                pltpu.make_async_copy(p_hbm.at[src_v.at[base + k]],
                                      bufs[k], gsem[k]).wait()
                pltpu.async_copy(bufs[k], agg_sh.at[dst_v.at[base + k]],
                                 ssem[k], add=True)
            for k in range(NBUF):
                nxt = base + NBUF + k
                pltpu.make_async_copy(bufs[k], agg_sh.at[dst_v.at[base + k]],
                                      ssem[k]).wait()
                pltpu.async_copy(p_hbm.at[src_v.at[nxt]], bufs[k], gsem[k])
            return carry2
        lax.fori_loop(0, IBLK // NBUF - 1, step, 0)

        last = IBLK - NBUF
        for k in range(NBUF):
            pltpu.make_async_copy(p_hbm.at[src_v.at[last + k]],
                                  bufs[k], gsem[k]).wait()
            pltpu.async_copy(bufs[k], agg_sh.at[dst_v.at[last + k]],
                             ssem[k], add=True)
        for k in range(NBUF):
            pltpu.make_async_copy(bufs[k], agg_sh.at[dst_v.at[last + k]],
                                  ssem[k]).wait()
        return carry
    lax.fori_loop(0, nib, blk, 0)

    plsc.subcore_barrier()

    def ocp(j, carry):
        r0 = zbase + j * CHUNK
        pltpu.sync_copy(agg_sh.at[pl.ds(r0, CHUNK)], rows0)
        pltpu.sync_copy(rows0, out_hbm.at[c, pl.ds(r0, CHUNK)])
        return carry
    lax.fori_loop(0, ACC_PER_SUB // CHUNK, ocp, 0)


def _sc_aggregate(p, src_r, dst_r):
    return pl.kernel(
        _agg_body,
        out_type=jax.ShapeDtypeStruct((NC, ACC_ROWS, D), jnp.float32),
        mesh=_mesh(),
        scratch_types=[
            pltpu.VMEM((IBLK, CHUNK), jnp.int32),
            pltpu.VMEM((IBLK, CHUNK), jnp.int32),
            pltpu.VMEM((CHUNK, D), jnp.float32),
            pltpu.VMEM((CHUNK, D), jnp.float32),
            pltpu.VMEM_SHARED((ACC_ROWS, D), jnp.float32),
            pltpu.SemaphoreType.DMA,
            pltpu.SemaphoreType.DMA,
            pltpu.SemaphoreType.DMA,
            pltpu.SemaphoreType.DMA,
        ],
    )(p, src_r, dst_r)


def _deg_body(dst_hbm, out_hbm, dst_v, ones_v, agg_sh):
    c = lax.axis_index("c")
    s = lax.axis_index("s")

    _zero_vmem(ones_v, CHUNK, D)
    zbase = s * ACC_PER_SUB
    def zcp(j, carry):
        pltpu.sync_copy(ones_v, agg_sh.at[pl.ds(zbase + j * CHUNK, CHUNK)])
        return carry
    lax.fori_loop(0, ACC_PER_SUB // CHUNK, zcp, 0)

    _fill_vmem(ones_v, CHUNK, D, 1.0)
    plsc.subcore_barrier()

    def blk(b, carry):
        pltpu.sync_copy(dst_hbm.at[c, s, pl.ds(b * IBLK, IBLK)], dst_v)
        def step(j, carry2):
            pltpu.sync_copy(ones_v, agg_sh.at[dst_v.at[j]], add=True)
            return carry2
        lax.fori_loop(0, IBLK, step, 0)
        return carry
    lax.fori_loop(0, NIB_DEG, blk, 0)

    plsc.subcore_barrier()

    def ocp(j, carry):
        r0 = zbase + j * CHUNK
        pltpu.sync_copy(agg_sh.at[pl.ds(r0, CHUNK)], ones_v)
        pltpu.sync_copy(ones_v, out_hbm.at[c, pl.ds(r0, CHUNK)])
        return carry
    lax.fori_loop(0, ACC_PER_SUB // CHUNK, ocp, 0)


def _sc_degrees(dst_r):
    return pl.kernel(
        _deg_body,
        out_type=jax.ShapeDtypeStruct((NC, ACC_ROWS, D), jnp.float32),
        mesh=_mesh(),
        scratch_types=[
            pltpu.VMEM((IBLK, CHUNK), jnp.int32),
            pltpu.VMEM((CHUNK, D), jnp.float32),
            pltpu.VMEM_SHARED((ACC_ROWS, D), jnp.float32),
        ],
    )(dst_r)


RB = 2000
GRID = N_NODES // RB


def _matmul_body(h_ref, w_ref, o_ref):
    o_ref[...] = jnp.dot(h_ref[...], w_ref[...],
                         preferred_element_type=jnp.float32)


def _tc_matmul(h, w):
    return pl.pallas_call(
        _matmul_body,
        grid=(GRID,),
        in_specs=[
            pl.BlockSpec((RB, D), lambda i: (i, 0)),
            pl.BlockSpec((D, D), lambda i: (0, 0)),
        ],
        out_specs=pl.BlockSpec((RB, D), lambda i: (i, 0)),
        out_shape=jax.ShapeDtypeStruct((N_NODES, D), jnp.float32),
    )(h, w)


def _combine_body(apply_relu, h_ref, w_ref, b_ref, parts_ref, deg_ref, o_ref):
    agg = parts_ref[0] + parts_ref[1]
    deg = deg_ref[0, :, 0:1] + deg_ref[1, :, 0:1]
    inv = 1.0 / jnp.maximum(deg, 1.0)
    out = jnp.dot(h_ref[...], w_ref[...],
                  preferred_element_type=jnp.float32)
    out = out + b_ref[...] + inv * agg
    if apply_relu:
        out = jnp.maximum(out, 0.0)
    o_ref[...] = out


def _tc_combine(h, w_self, b, parts, deg_parts, apply_relu):
    return pl.pallas_call(
        functools.partial(_combine_body, apply_relu),
        grid=(GRID,),
        in_specs=[
            pl.BlockSpec((RB, D), lambda i: (i, 0)),
            pl.BlockSpec((D, D), lambda i: (0, 0)),
            pl.BlockSpec((1, D), lambda i: (0, 0)),
            pl.BlockSpec((NC, RB, D), lambda i: (0, i, 0)),
            pl.BlockSpec((NC, RB, D), lambda i: (0, i, 0)),
        ],
        out_specs=pl.BlockSpec((RB, D), lambda i: (i, 0)),
        out_shape=jax.ShapeDtypeStruct((N_NODES, D), jnp.float32),
    )(h, w_self, b.reshape(1, D), parts, deg_parts)


def kernel(x, edge_index, W_self0, W_neigh0, b0, W_self1, W_neigh1, b1,
           W_self2, W_neigh2, b2):
    src = edge_index[0].astype(jnp.int32)
    dst = edge_index[1].astype(jnp.int32)
    pad = E_PAD - src.shape[0]
    arange_pad = jnp.arange(pad, dtype=jnp.int32) % CHUNK
    src_p = jnp.concatenate([src, jnp.zeros((pad,), jnp.int32)])
    dst_p = jnp.concatenate([dst, DUMP_ROW + arange_pad])

    n0 = NS * CH_C[0] * CHUNK
    ch1 = NIB_C[1] * IBLK
    filler_i = jnp.zeros((NS, CH_C[0] - ch1, CHUNK), jnp.int32)
    filler_d = jnp.full((NS, CH_C[0] - ch1, CHUNK), DUMP_ROW, jnp.int32)
    src_r = jnp.stack([
        src_p[:n0].reshape(NS, CH_C[0], CHUNK),
        jnp.concatenate(
            [src_p[n0:].reshape(NS, ch1, CHUNK), filler_i], axis=1)])
    dst_r = jnp.stack([
        dst_p[:n0].reshape(NS, CH_C[0], CHUNK),
        jnp.concatenate(
            [dst_p[n0:].reshape(NS, ch1, CHUNK), filler_d], axis=1)])

    dst_deg = dst_p.reshape(NC, NS, CH, CHUNK)
    deg_parts = _sc_degrees(dst_deg)

    h = x
    for (w_s, w_n, b, relu) in (
            (W_self0, W_neigh0, b0, True),
            (W_self1, W_neigh1, b1, True),
            (W_self2, W_neigh2, b2, False)):
        p = _tc_matmul(h, w_n)
        parts = _sc_aggregate(p, src_r, dst_r)
        h = _tc_combine(h, w_s, b, parts, deg_parts, relu)
    return h


# speedup vs baseline: 1.5391x; 1.1079x over previous
import functools

import jax
import jax.numpy as jnp
from jax import lax
from jax.experimental import pallas as pl
from jax.experimental.pallas import tpu as pltpu
from jax.experimental.pallas import tpu_sc as plsc

N_NODES = 10000
D = 128

NC = 2
NS = 16

CHUNK = 128
CH = 80
E_PAD = NC * NS * CH * CHUNK
IBLK = 16
NIB_C = (9, 1)
CH_C = (NIB_C[0] * IBLK, max(NIB_C[1], 1) * IBLK)
NIB_DEG = 5

ACC_ROWS = 10240
DUMP_ROW = N_NODES
ACC_PER_SUB = ACC_ROWS // NS

_mesh = lambda: plsc.VectorSubcoreMesh(core_axis_name="c", subcore_axis_name="s")


def _fill_vmem(buf, n_rows, width, value):
    def row(i, carry):
        for k in range(width // 16):
            buf[i, pl.ds(k * 16, 16)] = jnp.full((16,), value, jnp.float32)
        return carry
    lax.fori_loop(0, n_rows, row, 0)


def _zero_vmem(buf, n_rows, width):
    _fill_vmem(buf, n_rows, width, 0.0)


NBUF = 2


def _agg_body(p_hbm, src_hbm, dst_hbm, out_hbm,
              src_v, dst_v, rows0, rows1, agg_sh,
              gs0, gs1, ss0, ss1):
    c = lax.axis_index("c")
    s = lax.axis_index("s")
    bufs = (rows0, rows1)
    gsem = (gs0, gs1)
    ssem = (ss0, ss1)

    _zero_vmem(rows0, CHUNK, D)
    zbase = s * ACC_PER_SUB
    def zcp(j, carry):
        pltpu.sync_copy(rows0, agg_sh.at[pl.ds(zbase + j * CHUNK, CHUNK)])
        return carry
    lax.fori_loop(0, ACC_PER_SUB // CHUNK, zcp, 0)
    plsc.subcore_barrier()

    nib = jnp.where(c == 0, NIB_C[0], NIB_C[1])
    def blk(b, carry):
        pltpu.sync_copy(src_hbm.at[c, s, pl.ds(b * IBLK, IBLK)], src_v)
        pltpu.sync_copy(dst_hbm.at[c, s, pl.ds(b * IBLK, IBLK)], dst_v)

        for k in range(NBUF):
            pltpu.async_copy(p_hbm.at[src_v.at[k]], bufs[k], gsem[k])

        def step(jj, carry2):
            base = jj * NBUF
            for k in range(NBUF):
                pltpu.make_async_copy(p_hbm.at[src_v.at[base + k]],
                                      bufs[k], gsem[k]).wait()
                pltpu.async_copy(bufs[k], agg_sh.at[dst_v.at[base + k]],
                                 ssem[k], add=True)
            for k in range(NBUF):
                nxt = base + NBUF + k
                pltpu.make_async_copy(bufs[k], agg_sh.at[dst_v.at[base + k]],
                                      ssem[k]).wait()
                pltpu.async_copy(p_hbm.at[src_v.at[nxt]], bufs[k], gsem[k])
            return carry2
        lax.fori_loop(0, IBLK // NBUF - 1, step, 0)

        last = IBLK - NBUF
        for k in range(NBUF):
            pltpu.make_async_copy(p_hbm.at[src_v.at[last + k]],
                                  bufs[k], gsem[k]).wait()
            pltpu.async_copy(bufs[k], agg_sh.at[dst_v.at[last + k]],
                             ssem[k], add=True)
        for k in range(NBUF):
            pltpu.make_async_copy(bufs[k], agg_sh.at[dst_v.at[last + k]],
                                  ssem[k]).wait()
        return carry
    lax.fori_loop(0, nib, blk, 0)

    plsc.subcore_barrier()

    def ocp(j, carry):
        r0 = zbase + j * CHUNK
        pltpu.sync_copy(agg_sh.at[pl.ds(r0, CHUNK)], rows0)
        pltpu.sync_copy(rows0, out_hbm.at[c, pl.ds(r0, CHUNK)])
        return carry
    lax.fori_loop(0, ACC_PER_SUB // CHUNK, ocp, 0)


def _sc_aggregate(p, src_r, dst_r):
    return pl.kernel(
        _agg_body,
        out_type=jax.ShapeDtypeStruct((NC, ACC_ROWS, D), jnp.float32),
        mesh=_mesh(),
        scratch_types=[
            pltpu.VMEM((IBLK, CHUNK), jnp.int32),
            pltpu.VMEM((IBLK, CHUNK), jnp.int32),
            pltpu.VMEM((CHUNK, D), jnp.float32),
            pltpu.VMEM((CHUNK, D), jnp.float32),
            pltpu.VMEM_SHARED((ACC_ROWS, D), jnp.float32),
            pltpu.SemaphoreType.DMA,
            pltpu.SemaphoreType.DMA,
            pltpu.SemaphoreType.DMA,
            pltpu.SemaphoreType.DMA,
        ],
    )(p, src_r, dst_r)


def _deg_body(dst_hbm, out_hbm, dst_v, ones_v, agg_sh):
    c = lax.axis_index("c")
    s = lax.axis_index("s")

    _zero_vmem(ones_v, CHUNK, D)
    zbase = s * ACC_PER_SUB
    def zcp(j, carry):
        pltpu.sync_copy(ones_v, agg_sh.at[pl.ds(zbase + j * CHUNK, CHUNK)])
        return carry
    lax.fori_loop(0, ACC_PER_SUB // CHUNK, zcp, 0)

    _fill_vmem(ones_v, CHUNK, D, 1.0)
    plsc.subcore_barrier()

    def blk(b, carry):
        pltpu.sync_copy(dst_hbm.at[c, s, pl.ds(b * IBLK, IBLK)], dst_v)
        def step(j, carry2):
            pltpu.sync_copy(ones_v, agg_sh.at[dst_v.at[j]], add=True)
            return carry2
        lax.fori_loop(0, IBLK, step, 0)
        return carry
    lax.fori_loop(0, NIB_DEG, blk, 0)

    plsc.subcore_barrier()

    def ocp(j, carry):
        r0 = zbase + j * CHUNK
        pltpu.sync_copy(agg_sh.at[pl.ds(r0, CHUNK)], ones_v)
        pltpu.sync_copy(ones_v, out_hbm.at[c, pl.ds(r0, CHUNK)])
        return carry
    lax.fori_loop(0, ACC_PER_SUB // CHUNK, ocp, 0)


def _sc_degrees(dst_r):
    return pl.kernel(
        _deg_body,
        out_type=jax.ShapeDtypeStruct((NC, ACC_ROWS, D), jnp.float32),
        mesh=_mesh(),
        scratch_types=[
            pltpu.VMEM((IBLK, CHUNK), jnp.int32),
            pltpu.VMEM((CHUNK, D), jnp.float32),
            pltpu.VMEM_SHARED((ACC_ROWS, D), jnp.float32),
        ],
    )(dst_r)


RB = 2000
GRID = N_NODES // RB


def _matmul_body(h_ref, w_ref, o_ref):
    o_ref[...] = jnp.dot(h_ref[...], w_ref[...],
                         preferred_element_type=jnp.float32)


def _tc_matmul(h, w):
    return pl.pallas_call(
        _matmul_body,
        grid=(GRID,),
        in_specs=[
            pl.BlockSpec((RB, D), lambda i: (i, 0)),
            pl.BlockSpec((D, D), lambda i: (0, 0)),
        ],
        out_specs=pl.BlockSpec((RB, D), lambda i: (i, 0)),
        out_shape=jax.ShapeDtypeStruct((N_NODES, D), jnp.float32),
    )(h, w)


def _combine_body(apply_relu, h_ref, w_ref, b_ref, parts_ref, deg_ref, o_ref):
    agg = parts_ref[0] + parts_ref[1]
    deg = deg_ref[0, :, 0:1] + deg_ref[1, :, 0:1]
    inv = 1.0 / jnp.maximum(deg, 1.0)
    out = jnp.dot(h_ref[...], w_ref[...],
                  preferred_element_type=jnp.float32)
    out = out + b_ref[...] + inv * agg
    if apply_relu:
        out = jnp.maximum(out, 0.0)
    o_ref[...] = out


def _tc_combine(h, w_self, b, parts, deg_parts, apply_relu):
    return pl.pallas_call(
        functools.partial(_combine_body, apply_relu),
        grid=(GRID,),
        in_specs=[
            pl.BlockSpec((RB, D), lambda i: (i, 0)),
            pl.BlockSpec((D, D), lambda i: (0, 0)),
            pl.BlockSpec((1, D), lambda i: (0, 0)),
            pl.BlockSpec((NC, RB, D), lambda i: (0, i, 0)),
            pl.BlockSpec((NC, RB, D), lambda i: (0, i, 0)),
        ],
        out_specs=pl.BlockSpec((RB, D), lambda i: (i, 0)),
        out_shape=jax.ShapeDtypeStruct((N_NODES, D), jnp.float32),
    )(h, w_self, b.reshape(1, D), parts, deg_parts)


def kernel(x, edge_index, W_self0, W_neigh0, b0, W_self1, W_neigh1, b1,
           W_self2, W_neigh2, b2):
    src = edge_index[0].astype(jnp.int32)
    dst = edge_index[1].astype(jnp.int32)
    pad = E_PAD - src.shape[0]
    arange_pad = jnp.arange(pad, dtype=jnp.int32) % CHUNK
    src_p = jnp.concatenate([src, jnp.zeros((pad,), jnp.int32)])
    dst_p = jnp.concatenate([dst, DUMP_ROW + arange_pad])

    n0 = NS * CH_C[0] * CHUNK
    ch1 = NIB_C[1] * IBLK
    filler_i = jnp.zeros((NS, CH_C[0] - ch1, CHUNK), jnp.int32)
    filler_d = jnp.full((NS, CH_C[0] - ch1, CHUNK), DUMP_ROW, jnp.int32)
    src_r = jnp.stack([
        src_p[:n0].reshape(NS, CH_C[0], CHUNK),
        jnp.concatenate(
            [src_p[n0:].reshape(NS, ch1, CHUNK), filler_i], axis=1)])
    dst_r = jnp.stack([
        dst_p[:n0].reshape(NS, CH_C[0], CHUNK),
        jnp.concatenate(
            [dst_p[n0:].reshape(NS, ch1, CHUNK), filler_d], axis=1)])

    dst_deg = dst_p.reshape(NC, NS, CH, CHUNK)
    deg_parts = _sc_degrees(dst_deg)

    h = x
    for (w_s, w_n, b, relu) in (
            (W_self0, W_neigh0, b0, True),
            (W_self1, W_neigh1, b1, True),
            (W_self2, W_neigh2, b2, False)):
        p = _tc_matmul(h, w_n)
        parts = _sc_aggregate(p, src_r, dst_r)
        h = _tc_combine(h, w_s, b, parts, deg_parts, relu)
    return h
